# Initial kernel scaffold; baseline (speedup 1.0000x reference)
import jax
import jax.numpy as jnp
from jax.experimental import pallas as pl


def kernel(X, force):
    raise NotImplementedError("write your pallas kernel here")


# speedup vs baseline: 4.8814x; 4.8814x over previous
import functools

import jax
import jax.numpy as jnp
from jax import lax
from jax.experimental import pallas as pl
from jax.experimental.pallas import tpu as pltpu
from jax.experimental.pallas import tpu_sc as plsc

_N = 1024
_LOWER = -4.0
_UPPER = 4.0
_DT = 0.01
_B = 131072

_NC = 2
_NS = 16
_NW = _NC * _NS
_L = 16
_CHUNK = (2 * _B) // _NW
_STEPS = _CHUNK // _L

_mesh = plsc.VectorSubcoreMesh(core_axis_name="c", subcore_axis_name="s")


@functools.partial(
    pl.kernel,
    out_type=jax.ShapeDtypeStruct((2 * _B,), jnp.float32),
    mesh=_mesh,
    scratch_types=[
        pltpu.VMEM((_CHUNK,), jnp.float32),
        pltpu.VMEM((_N,), jnp.float32),
    ],
    compiler_params=pltpu.CompilerParams(needs_layout_passes=False),
)
def _sc_kernel(x_hbm, force_hbm, out_hbm, buf, force_v):
    wid = lax.axis_index("s") * _NC + lax.axis_index("c")
    base = wid * _CHUNK
    pltpu.sync_copy(force_hbm, force_v)
    pltpu.sync_copy(x_hbm.at[pl.ds(base, _CHUNK)], buf)

    lanes = lax.iota(jnp.int32, _L)
    even = lanes - (lanes % 2)
    odd_dt = jnp.where(lanes % 2 == 1, _DT, 0.0)
    scale = _N / (_UPPER - _LOWER)

    def step(j, carry):
        pair = buf[pl.ds(j * _L, _L)]
        u = (pair - _LOWER) * scale
        u = jnp.minimum(jnp.maximum(u, 0.0), float(_N - 1))
        idx = (u + 0.5).astype(jnp.int32)
        idx2 = idx.at[even].get(mode="promise_in_bounds")
        f = plsc.load_gather(force_v, [idx2])
        buf[pl.ds(j * _L, _L)] = pair + f * odd_dt
        return carry

    lax.fori_loop(0, _STEPS, step, 0)
    pltpu.sync_copy(buf, out_hbm.at[pl.ds(base, _CHUNK)])


def kernel(X, force):
    flat = _sc_kernel(X.reshape(2 * _B), force)
    return flat.reshape(_B, 2)


# speedup vs baseline: 5.0396x; 1.0324x over previous
import functools

import jax
import jax.numpy as jnp
from jax import lax
from jax.experimental import pallas as pl
from jax.experimental.pallas import tpu as pltpu
from jax.experimental.pallas import tpu_sc as plsc

_N = 1024
_LOWER = -4.0
_UPPER = 4.0
_DT = 0.01
_B = 131072

_NC = 2
_NS = 16
_NW = _NC * _NS
_L = 16
_CHUNK = (2 * _B) // _NW
_STEPS = _CHUNK // _L

_mesh = plsc.VectorSubcoreMesh(core_axis_name="c", subcore_axis_name="s")


@functools.partial(
    pl.kernel,
    out_type=jax.ShapeDtypeStruct((2 * _B,), jnp.float32),
    mesh=_mesh,
    scratch_types=[
        pltpu.VMEM((_CHUNK,), jnp.float32),
        pltpu.VMEM((_N,), jnp.float32),
    ],
    compiler_params=pltpu.CompilerParams(needs_layout_passes=False),
)
def _sc_kernel(x_hbm, force_hbm, out_hbm, buf, force_v):
    wid = lax.axis_index("s") * _NC + lax.axis_index("c")
    base = wid * _CHUNK
    pltpu.sync_copy(force_hbm, force_v)
    pltpu.sync_copy(x_hbm.at[pl.ds(base, _CHUNK)], buf)

    lanes = lax.iota(jnp.int32, _L)
    even = lanes - (lanes % 2)
    odd_dt = jnp.where(lanes % 2 == 1, _DT, 0.0)
    scale = _N / (_UPPER - _LOWER)

    def step(j, carry):
        pair = buf[pl.ds(j * _L, _L)]
        u = pair * scale + (0.5 - _LOWER * scale)
        u = jnp.minimum(jnp.maximum(u, 0.5), float(_N - 1) + 0.5)
        idx = u.astype(jnp.int32)
        idx2 = idx.at[even].get(mode="promise_in_bounds")
        f = plsc.load_gather(force_v, [idx2])
        buf[pl.ds(j * _L, _L)] = pair + f * odd_dt
        return carry

    lax.fori_loop(0, _STEPS, step, 0, unroll=8)
    pltpu.sync_copy(buf, out_hbm.at[pl.ds(base, _CHUNK)])


def kernel(X, force):
    flat = _sc_kernel(X.reshape(2 * _B), force)
    return flat.reshape(_B, 2)
